# baseline (speedup 1.0000x reference)
import functools

import jax
import jax.numpy as jnp
from jax import lax
from jax.experimental import pallas as pl
from jax.experimental.pallas import tpu as pltpu
from jax.experimental.pallas import tpu_sc as plsc

B = 64
C = 3
IMG = 512
P = 16
GRID = IMG // P
NUM_PATCHES = GRID * GRID
L = P * P
KEEP = 25
PAD = 32

NC = 2
NS = 16
NW = NC * NS
UNITS = B * GRID
UNITS_PER_W = UNITS // NW


def _vsort(k, descending):
    ks, _ = plsc.sort_key_val(k, k, descending=descending)
    return ks


@functools.partial(
    pl.kernel,
    mesh=plsc.VectorSubcoreMesh(core_axis_name="c", subcore_axis_name="s"),
    compiler_params=pltpu.CompilerParams(needs_layout_passes=False),
    out_type=(
        jax.ShapeDtypeStruct((B, C, NUM_PATCHES, KEEP), jnp.float32),
        jax.ShapeDtypeStruct((B, C, NUM_PATCHES, KEEP), jnp.int32),
    ),
    scratch_types=[
        pltpu.VMEM((2, GRID, L), jnp.float32),
        pltpu.VMEM((2, C * P, IMG), jnp.float32),
        pltpu.VMEM((2, C, GRID, KEEP), jnp.float32),
        pltpu.VMEM((2, GRID, KEEP), jnp.int32),
        pltpu.VMEM((4, PAD), jnp.int32),
        pltpu.SemaphoreType.DMA((2,)),
        pltpu.SemaphoreType.DMA((2,)),
    ],
)
def _patch_embed_sc(x_hbm, noise_hbm, out_hbm, ids_hbm, nz, xs, ov, oi, tid,
                    sem_in, sem_out):
    wid = lax.axis_index("s") * NC + lax.axis_index("c")

    def in_copies(t, par):
        u = wid * UNITS_PER_W + t
        b = u // GRID
        gy = u % GRID
        cps = [pltpu.make_async_copy(
            noise_hbm.at[b, pl.ds(gy * GRID, GRID), :], nz.at[par],
            sem_in.at[par])]
        for c in range(C):
            cps.append(pltpu.make_async_copy(
                x_hbm.at[b, c, pl.ds(gy * P, P), :],
                xs.at[par, pl.ds(c * P, P), :], sem_in.at[par]))
        return cps

    def out_copies(t, par):
        u = wid * UNITS_PER_W + t
        b = u // GRID
        gy = u % GRID
        cps = []
        for c in range(C):
            cps.append(pltpu.make_async_copy(
                ov.at[par, c], out_hbm.at[b, c, pl.ds(gy * GRID, GRID), :],
                sem_out.at[par]))
            cps.append(pltpu.make_async_copy(
                oi.at[par], ids_hbm.at[b, c, pl.ds(gy * GRID, GRID), :],
                sem_out.at[par]))
        return cps

    def topk_patch(par, j, slot):
        leaves = []
        for i in range(P):
            v = nz[par, j, pl.ds(i * P, P)]
            q = (v * 8388608.0).astype(jnp.int32)
            key = jnp.left_shift(q, 8) | (lax.iota(jnp.int32, P) + i * P)
            key = lax.bitcast_convert_type(key, jnp.uint32)
            leaves.append(_vsort(key, descending=(i % 2 == 1)))
        lists = []
        for p in range(8):
            lo = jnp.minimum(leaves[2 * p], leaves[2 * p + 1])
            hi = jnp.maximum(leaves[2 * p], leaves[2 * p + 1])
            if p % 2 == 0:
                lists.append((_vsort(lo, False), _vsort(hi, False)))
            else:
                lists.append((_vsort(hi, True), _vsort(lo, True)))
        while len(lists) > 1:
            nxt = []
            for p in range(len(lists) // 2):
                a0, a1 = lists[2 * p]
                b0, b1 = lists[2 * p + 1]
                w0 = jnp.minimum(a0, b0)
                w1 = jnp.minimum(a1, b1)
                lo = jnp.minimum(w0, w1)
                hi = jnp.maximum(w0, w1)
                if p % 2 == 0:
                    nxt.append((_vsort(lo, False), _vsort(hi, False)))
                else:
                    nxt.append((_vsort(hi, True), _vsort(lo, True)))
            lists = nxt
        k0, k1 = lists[0]

        id0 = lax.bitcast_convert_type(k0, jnp.int32) & 255
        id1 = lax.bitcast_convert_type(k1, jnp.int32) & 255
        tid[slot, pl.ds(0, P)] = id0
        tid[slot, pl.ds(P, P)] = id1
        idb = tid[slot, pl.ds(9, P)]
        oi[par, j, pl.ds(0, P)] = id0
        oi[par, j, pl.ds(9, P)] = idb

        row0 = jnp.right_shift(id0, 4)
        rowb = jnp.right_shift(idb, 4)
        col0 = j * P + (id0 & 15)
        colb = j * P + (idb & 15)
        par_v = jnp.broadcast_to(par, (P,))
        for c in range(C):
            ov[par, c, j, pl.ds(0, P)] = plsc.load_gather(
                xs, [par_v, row0 + c * P, col0])
            ov[par, c, j, pl.ds(9, P)] = plsc.load_gather(
                xs, [par_v, rowb + c * P, colb])

    for cp in in_copies(0, 0):
        cp.start()

    def unit_body(t, carry):
        par = lax.rem(t, 2)

        @pl.when(t + 1 < UNITS_PER_W)
        def _prefetch():
            for cp in in_copies(t + 1, lax.rem(t + 1, 2)):
                cp.start()

        @pl.when(t >= 2)
        def _drain_out():
            for cp in out_copies(t - 2, par):
                cp.wait()

        for cp in in_copies(t, par):
            cp.wait()

        def patch_body(jj, pcarry):
            for k in range(4):
                topk_patch(par, 4 * jj + k, k)
            return pcarry

        lax.fori_loop(0, GRID // 4, patch_body, 0)

        for cp in out_copies(t, par):
            cp.start()
        return carry

    lax.fori_loop(0, UNITS_PER_W, unit_body, 0)

    for cp in out_copies(UNITS_PER_W - 2, 0):
        cp.wait()
    for cp in out_copies(UNITS_PER_W - 1, 1):
        cp.wait()


def kernel(x, noise):
    return _patch_embed_sc(x, noise)


# speedup vs baseline: 1.2334x; 1.2334x over previous
import functools

import jax
import jax.numpy as jnp
from jax import lax
from jax.experimental import pallas as pl
from jax.experimental.pallas import tpu as pltpu
from jax.experimental.pallas import tpu_sc as plsc

B = 64
C = 3
IMG = 512
P = 16
GRID = IMG // P
NUM_PATCHES = GRID * GRID
L = P * P
KEEP = 25
PAD = 32

NC = 2
NS = 16
NW = NC * NS
UNITS = B * GRID
UNITS_PER_W = UNITS // NW


def _vsort(k, descending):
    ks, _ = plsc.sort_key_val(k, k, descending=descending)
    return ks


@functools.partial(
    pl.kernel,
    mesh=plsc.VectorSubcoreMesh(core_axis_name="c", subcore_axis_name="s"),
    compiler_params=pltpu.CompilerParams(needs_layout_passes=False),
    out_type=(
        jax.ShapeDtypeStruct((B, C, NUM_PATCHES, PAD), jnp.float32),
        jax.ShapeDtypeStruct((B, C, NUM_PATCHES, PAD), jnp.int32),
    ),
    scratch_types=[
        pltpu.VMEM((2, GRID, L), jnp.float32),
        pltpu.VMEM((2, C * P, IMG), jnp.float32),
        pltpu.VMEM((2, C, GRID, PAD), jnp.float32),
        pltpu.VMEM((2, GRID, PAD), jnp.int32),
        pltpu.SemaphoreType.DMA((2,)),
        pltpu.SemaphoreType.DMA((2,)),
    ],
)
def _patch_embed_sc(x_hbm, noise_hbm, out_hbm, ids_hbm, nz, xs, ov, oi,
                    sem_in, sem_out):
    wid = lax.axis_index("s") * NC + lax.axis_index("c")

    def in_copies(t, par):
        u = wid * UNITS_PER_W + t
        b = u // GRID
        gy = u % GRID
        cps = [pltpu.make_async_copy(
            noise_hbm.at[b, pl.ds(gy * GRID, GRID), :], nz.at[par],
            sem_in.at[par])]
        for c in range(C):
            cps.append(pltpu.make_async_copy(
                x_hbm.at[b, c, pl.ds(gy * P, P), :],
                xs.at[par, pl.ds(c * P, P), :], sem_in.at[par]))
        return cps

    def out_copies(t, par):
        u = wid * UNITS_PER_W + t
        b = u // GRID
        gy = u % GRID
        cps = []
        for c in range(C):
            cps.append(pltpu.make_async_copy(
                ov.at[par, c], out_hbm.at[b, c, pl.ds(gy * GRID, GRID), :],
                sem_out.at[par]))
            cps.append(pltpu.make_async_copy(
                oi.at[par], ids_hbm.at[b, c, pl.ds(gy * GRID, GRID), :],
                sem_out.at[par]))
        return cps

    def topk_patch(par, j):
        leaves = []
        for i in range(P):
            v = nz[par, j, pl.ds(i * P, P)]
            q = (v * 8388608.0).astype(jnp.int32)
            key = jnp.left_shift(q, 8) | (lax.iota(jnp.int32, P) + i * P)
            key = lax.bitcast_convert_type(key, jnp.uint32)
            leaves.append(_vsort(key, descending=(i % 2 == 1)))
        lists = []
        for p in range(8):
            lo = jnp.minimum(leaves[2 * p], leaves[2 * p + 1])
            hi = jnp.maximum(leaves[2 * p], leaves[2 * p + 1])
            if p % 2 == 0:
                lists.append((_vsort(lo, False), _vsort(hi, False)))
            else:
                lists.append((_vsort(hi, True), _vsort(lo, True)))
        while len(lists) > 1:
            nxt = []
            for p in range(len(lists) // 2):
                a0, a1 = lists[2 * p]
                b0, b1 = lists[2 * p + 1]
                w0 = jnp.minimum(a0, b0)
                w1 = jnp.minimum(a1, b1)
                lo = jnp.minimum(w0, w1)
                hi = jnp.maximum(w0, w1)
                if p % 2 == 0:
                    nxt.append((_vsort(lo, False), _vsort(hi, False)))
                else:
                    nxt.append((_vsort(hi, True), _vsort(lo, True)))
            lists = nxt
        k0, k1 = lists[0]

        id0 = lax.bitcast_convert_type(k0, jnp.int32) & 255
        id1 = lax.bitcast_convert_type(k1, jnp.int32) & 255
        oi[par, j, pl.ds(0, P)] = id0
        oi[par, j, pl.ds(P, P)] = id1

        row0 = jnp.right_shift(id0, 4)
        row1 = jnp.right_shift(id1, 4)
        col0 = j * P + (id0 & 15)
        col1 = j * P + (id1 & 15)
        par_v = jnp.broadcast_to(par, (P,))
        for c in range(C):
            ov[par, c, j, pl.ds(0, P)] = plsc.load_gather(
                xs, [par_v, row0 + c * P, col0])
            ov[par, c, j, pl.ds(P, P)] = plsc.load_gather(
                xs, [par_v, row1 + c * P, col1])

    for cp in in_copies(0, 0):
        cp.start()

    def unit_body(t, carry):
        par = lax.rem(t, 2)

        @pl.when(t + 1 < UNITS_PER_W)
        def _prefetch():
            for cp in in_copies(t + 1, lax.rem(t + 1, 2)):
                cp.start()

        @pl.when(t >= 2)
        def _drain_out():
            for cp in out_copies(t - 2, par):
                cp.wait()

        for cp in in_copies(t, par):
            cp.wait()

        def patch_body(jj, pcarry):
            for k in range(4):
                topk_patch(par, 4 * jj + k)
            return pcarry

        lax.fori_loop(0, GRID // 4, patch_body, 0)

        for cp in out_copies(t, par):
            cp.start()
        return carry

    lax.fori_loop(0, UNITS_PER_W, unit_body, 0)

    for cp in out_copies(UNITS_PER_W - 2, 0):
        cp.wait()
    for cp in out_copies(UNITS_PER_W - 1, 1):
        cp.wait()


def kernel(x, noise):
    vals, ids = _patch_embed_sc(x, noise)
    return vals[..., :KEEP], ids[..., :KEEP]
